# Initial kernel scaffold; baseline (speedup 1.0000x reference)
import jax
import jax.numpy as jnp
from jax.experimental import pallas as pl


def kernel(R, Z, neighbor_idx, radial_emb, W0, b0, W1, b1, W2, b2, scale, shift):
    raise NotImplementedError("write your pallas kernel here")


# speedup vs baseline: 8.1058x; 8.1058x over previous
import functools

import jax
import jax.numpy as jnp
import numpy as np
from jax.experimental import pallas as pl
from jax.experimental.pallas import tpu as pltpu

_N_BASIS = 7
_N_RADIAL = 5
_R_MIN = 0.5
_R_MAX = 6.0
_MOM = 40
_MOMP = 256
_E_BLK = 6400
_A_BLK = 1000


def _edge_kernel(idx_ref, dr_ref, cf_ref, out_ref, scr_ref):
    pid = pl.program_id(0)

    @pl.when(pid == 0)
    def _():
        out_ref[...] = jnp.zeros(out_ref.shape, out_ref.dtype)

    drx = dr_ref[0:1, :]
    dry = dr_ref[1:2, :]
    drz = dr_ref[2:3, :]
    dr2 = drx * drx + dry * dry + drz * drz
    dist = jnp.sqrt(dr2 + 1e-12)
    inv = 1.0 / dist
    ex = drx * inv
    ey = dry * inv
    ez = drz * inv

    betta = float(_N_BASIS) ** 2 / _R_MAX ** 2
    rad_norm = (2.0 * betta / np.pi) ** 0.25
    shifts = [_R_MIN + (_R_MAX - _R_MIN) / _N_BASIS * k for k in range(_N_BASIS)]
    cutoff = jnp.where(dist < _R_MAX,
                       0.5 * (jnp.cos(np.pi / _R_MAX * dist) + 1.0), 0.0)
    basis = [rad_norm * jnp.exp(-betta * (dist - s) ** 2) for s in shifts]

    radial = []
    for r in range(_N_RADIAL):
        acc = cf_ref[r * _N_BASIS:r * _N_BASIS + 1, :] * basis[0]
        for k in range(1, _N_BASIS):
            acc = acc + cf_ref[r * _N_BASIS + k:r * _N_BASIS + k + 1, :] * basis[k]
        radial.append(acc * cutoff)

    e = [ex, ey, ez]
    t = [jnp.ones_like(ex)] + e
    for i in range(3):
        for j in range(3):
            t.append(e[i] * e[j])
    for i in range(3):
        for j in range(3):
            for k in range(3):
                t.append(t[4 + 3 * i + j] * e[k])

    rows = []
    for r in range(_N_RADIAL):
        for c in range(_MOM):
            rows.append(radial[r] * t[c])
    pad = jnp.zeros((_MOMP - _N_RADIAL * _MOM, drx.shape[1]), dtype=jnp.float32)
    contrib = jnp.concatenate(rows + [pad], axis=0)
    scr_ref[...] = contrib.T

    def body(b, carry):
        a = idx_ref[0, 0, b]
        row = scr_ref[pl.ds(b, 1), :]
        out_ref[pl.ds(a, 1), :] = out_ref[pl.ds(a, 1), :] + row
        return carry

    jax.lax.fori_loop(0, dr_ref.shape[1], body, 0)


def _mlp_kernel(g_ref, w0_ref, b0_ref, w1_ref, b1_ref, w2_ref, b2_ref, o_ref,
                *, inv0, inv1):
    h = jnp.dot(g_ref[...], w0_ref[...],
                preferred_element_type=jnp.float32) * inv0 + 0.1 * b0_ref[...]
    h = h * jax.nn.sigmoid(h)
    h = jnp.dot(h, w1_ref[...],
                preferred_element_type=jnp.float32) * inv1 + 0.1 * b1_ref[...]
    h = h * jax.nn.sigmoid(h)
    o_ref[...] = jnp.dot(h, w2_ref[...],
                         preferred_element_type=jnp.float32) * inv1 + 0.1 * b2_ref[...]


def _tril2(n):
    i, j = np.tril_indices(n)
    return jnp.asarray(i), jnp.asarray(j)


def _tril3(n):
    idx = [(i, j, k) for i in range(n) for j in range(i + 1) for k in range(j + 1)]
    a = np.array(idx, dtype=np.int32)
    return jnp.asarray(a[:, 0]), jnp.asarray(a[:, 1]), jnp.asarray(a[:, 2])


@jax.jit
def kernel(R, Z, neighbor_idx, radial_emb, W0, b0, W1, b1, W2, b2, scale, shift):
    n_atoms = R.shape[0]
    n_edges = neighbor_idx.shape[1]
    idx_i = neighbor_idx[0]
    idx_j = neighbor_idx[1]

    dr = (R[idx_i] - R[idx_j]).T
    drp = jnp.concatenate(
        [dr, jnp.zeros((5, n_edges), dtype=jnp.float32)], axis=0)
    coeffs = radial_emb[Z[idx_i], Z[idx_j]].reshape(n_edges, -1).T
    cfp = jnp.concatenate(
        [coeffs, jnp.zeros((5, n_edges), dtype=jnp.float32)], axis=0)

    e_blk = _E_BLK if n_edges % _E_BLK == 0 else n_edges
    nblk = n_edges // e_blk
    idx3d = idx_i.astype(jnp.int32).reshape(nblk, 1, e_blk)

    mom = pl.pallas_call(
        _edge_kernel,
        grid=(nblk,),
        in_specs=[
            pl.BlockSpec((1, 1, e_blk), lambda i: (i, 0, 0),
                         memory_space=pltpu.SMEM),
            pl.BlockSpec((8, e_blk), lambda i: (0, i)),
            pl.BlockSpec((40, e_blk), lambda i: (0, i)),
        ],
        out_specs=pl.BlockSpec((n_atoms, _MOMP), lambda i: (0, 0)),
        out_shape=jax.ShapeDtypeStruct((n_atoms, _MOMP), jnp.float32),
        scratch_shapes=[pltpu.VMEM((e_blk, _MOMP), jnp.float32)],
    )(idx3d, drp, cfp)

    m = mom[:, :_N_RADIAL * _MOM].reshape(n_atoms, _N_RADIAL, _MOM)
    m0 = m[:, :, 0]
    m1 = m[:, :, 1:4]
    m2 = m[:, :, 4:13].reshape(n_atoms, _N_RADIAL, 3, 3)
    m3 = m[:, :, 13:40].reshape(n_atoms, _N_RADIAL, 3, 3, 3)

    contr_1 = jnp.einsum('ari,asi->ars', m1, m1)
    contr_2 = jnp.einsum('arij,asij->ars', m2, m2)
    contr_3 = jnp.einsum('arijk,asijk->ars', m3, m3)
    contr_4 = jnp.einsum('arij,asik,atjk->arst', m2, m2, m2)
    contr_5 = jnp.einsum('ari,asj,atij->arst', m1, m1, m2)
    contr_6 = jnp.einsum('arijk,asijl,atkl->arst', m3, m3, m2)
    contr_7 = jnp.einsum('arijk,asij,atk->arst', m3, m2, m1)
    i2, j2 = _tril2(_N_RADIAL)
    i3, j3, k3 = _tril3(_N_RADIAL)
    n = _N_RADIAL
    n2 = i2.shape[0]
    gm = jnp.concatenate([
        m0,
        contr_1[:, i2, j2],
        contr_2[:, i2, j2],
        contr_3[:, i2, j2],
        contr_4[:, i3, j3, k3],
        contr_5[:, i2, j2, :].reshape(-1, n2 * n),
        contr_6[:, i2, j2, :].reshape(-1, n2 * n),
        contr_7.reshape(-1, n * n * n),
    ], axis=-1)

    feat = gm.shape[-1]
    featp = 384
    gmp = jnp.concatenate(
        [gm, jnp.zeros((n_atoms, featp - feat), dtype=jnp.float32)], axis=1)
    w0p = jnp.concatenate(
        [W0, jnp.zeros((featp - feat, W0.shape[1]), dtype=jnp.float32)], axis=0)
    w2p = jnp.concatenate(
        [W2, jnp.zeros((W2.shape[0], 127), dtype=jnp.float32)], axis=1)
    b2p = jnp.concatenate([b2, jnp.zeros((127,), dtype=jnp.float32)])

    a_blk = _A_BLK if n_atoms % _A_BLK == 0 else n_atoms
    grid_b = n_atoms // a_blk
    hid = W0.shape[1]
    mlp = functools.partial(_mlp_kernel,
                            inv0=1.0 / np.sqrt(float(feat)),
                            inv1=1.0 / np.sqrt(float(hid)))
    h = pl.pallas_call(
        mlp,
        grid=(grid_b,),
        in_specs=[
            pl.BlockSpec((a_blk, featp), lambda i: (i, 0)),
            pl.BlockSpec((featp, hid), lambda i: (0, 0)),
            pl.BlockSpec((1, hid), lambda i: (0, 0)),
            pl.BlockSpec((hid, hid), lambda i: (0, 0)),
            pl.BlockSpec((1, hid), lambda i: (0, 0)),
            pl.BlockSpec((hid, 128), lambda i: (0, 0)),
            pl.BlockSpec((1, 128), lambda i: (0, 0)),
        ],
        out_specs=pl.BlockSpec((a_blk, 128), lambda i: (i, 0)),
        out_shape=jax.ShapeDtypeStruct((n_atoms, 128), jnp.float32),
    )(gmp, w0p, b0.reshape(1, hid), W1, b1.reshape(1, hid), w2p,
      b2p.reshape(1, 128))

    out = h[:, :1]
    out = scale[Z] * out + shift[Z]
    return jnp.where((Z > 0)[:, None], out, 0.0)


# speedup vs baseline: 8.6252x; 1.0641x over previous
import functools

import jax
import jax.numpy as jnp
import numpy as np
from jax.experimental import pallas as pl
from jax.experimental.pallas import tpu as pltpu

_N_BASIS = 7
_N_RADIAL = 5
_R_MIN = 0.5
_R_MAX = 6.0
_MOM = 40
_MOMP = 256
_E_BLK = 6400
_A_BLK = 1000


def _edge_kernel(idx_ref, dr_ref, cf_ref, out_ref, scr_ref, acc_ref):
    pid = pl.program_id(0)
    n_atoms = out_ref.shape[0]

    @pl.when(pid == 0)
    def _():
        acc_ref[...] = jnp.zeros(acc_ref.shape, acc_ref.dtype)

    drx = dr_ref[0:1, :]
    dry = dr_ref[1:2, :]
    drz = dr_ref[2:3, :]
    dr2 = drx * drx + dry * dry + drz * drz
    dist = jnp.sqrt(dr2 + 1e-12)
    inv = 1.0 / dist
    ex = drx * inv
    ey = dry * inv
    ez = drz * inv

    betta = float(_N_BASIS) ** 2 / _R_MAX ** 2
    rad_norm = (2.0 * betta / np.pi) ** 0.25
    shifts = [_R_MIN + (_R_MAX - _R_MIN) / _N_BASIS * k for k in range(_N_BASIS)]
    cutoff = jnp.where(dist < _R_MAX,
                       0.5 * (jnp.cos(np.pi / _R_MAX * dist) + 1.0), 0.0)
    basis = [rad_norm * jnp.exp(-betta * (dist - s) ** 2) for s in shifts]

    radial = []
    for r in range(_N_RADIAL):
        acc = cf_ref[r * _N_BASIS:r * _N_BASIS + 1, :] * basis[0]
        for k in range(1, _N_BASIS):
            acc = acc + cf_ref[r * _N_BASIS + k:r * _N_BASIS + k + 1, :] * basis[k]
        radial.append(acc * cutoff)

    e = [ex, ey, ez]
    t = [jnp.ones_like(ex)] + e
    for i in range(3):
        for j in range(3):
            t.append(e[i] * e[j])
    for i in range(3):
        for j in range(3):
            for k in range(3):
                t.append(t[4 + 3 * i + j] * e[k])

    rows = []
    for r in range(_N_RADIAL):
        for c in range(_MOM):
            rows.append(radial[r] * t[c])
    pad = jnp.zeros((_MOMP - _N_RADIAL * _MOM, drx.shape[1]), dtype=jnp.float32)
    contrib = jnp.concatenate(rows + [pad], axis=0)
    scr_ref[...] = contrib.T

    def body(b, carry):
        a0 = idx_ref[0, 0, 2 * b]
        a1 = idx_ref[0, 0, 2 * b + 1] + n_atoms
        acc_ref[pl.ds(a0, 1), :] = (acc_ref[pl.ds(a0, 1), :]
                                    + scr_ref[pl.ds(2 * b, 1), :])
        acc_ref[pl.ds(a1, 1), :] = (acc_ref[pl.ds(a1, 1), :]
                                    + scr_ref[pl.ds(2 * b + 1, 1), :])
        return carry

    jax.lax.fori_loop(0, dr_ref.shape[1] // 2, body, 0, unroll=4)

    @pl.when(pid == pl.num_programs(0) - 1)
    def _():
        out_ref[...] = acc_ref[0:n_atoms, :] + acc_ref[n_atoms:2 * n_atoms, :]


def _mlp_kernel(g_ref, w0_ref, b0_ref, w1_ref, b1_ref, w2_ref, b2_ref, o_ref,
                *, inv0, inv1):
    h = jnp.dot(g_ref[...], w0_ref[...],
                preferred_element_type=jnp.float32) * inv0 + 0.1 * b0_ref[...]
    h = h * jax.nn.sigmoid(h)
    h = jnp.dot(h, w1_ref[...],
                preferred_element_type=jnp.float32) * inv1 + 0.1 * b1_ref[...]
    h = h * jax.nn.sigmoid(h)
    o_ref[...] = jnp.dot(h, w2_ref[...],
                         preferred_element_type=jnp.float32) * inv1 + 0.1 * b2_ref[...]


def _tril2(n):
    i, j = np.tril_indices(n)
    return jnp.asarray(i), jnp.asarray(j)


def _tril3(n):
    idx = [(i, j, k) for i in range(n) for j in range(i + 1) for k in range(j + 1)]
    a = np.array(idx, dtype=np.int32)
    return jnp.asarray(a[:, 0]), jnp.asarray(a[:, 1]), jnp.asarray(a[:, 2])


@jax.jit
def kernel(R, Z, neighbor_idx, radial_emb, W0, b0, W1, b1, W2, b2, scale, shift):
    n_atoms = R.shape[0]
    n_edges = neighbor_idx.shape[1]
    idx_i = neighbor_idx[0]
    idx_j = neighbor_idx[1]

    dr = (R[idx_i] - R[idx_j]).T
    drp = jnp.concatenate(
        [dr, jnp.zeros((5, n_edges), dtype=jnp.float32)], axis=0)
    coeffs = radial_emb[Z[idx_i], Z[idx_j]].reshape(n_edges, -1).T
    cfp = jnp.concatenate(
        [coeffs, jnp.zeros((5, n_edges), dtype=jnp.float32)], axis=0)

    e_blk = _E_BLK if n_edges % _E_BLK == 0 else n_edges
    nblk = n_edges // e_blk
    idx3d = idx_i.astype(jnp.int32).reshape(nblk, 1, e_blk)

    mom = pl.pallas_call(
        _edge_kernel,
        grid=(nblk,),
        in_specs=[
            pl.BlockSpec((1, 1, e_blk), lambda i: (i, 0, 0),
                         memory_space=pltpu.SMEM),
            pl.BlockSpec((8, e_blk), lambda i: (0, i)),
            pl.BlockSpec((40, e_blk), lambda i: (0, i)),
        ],
        out_specs=pl.BlockSpec((n_atoms, _MOMP), lambda i: (0, 0)),
        out_shape=jax.ShapeDtypeStruct((n_atoms, _MOMP), jnp.float32),
        scratch_shapes=[pltpu.VMEM((e_blk, _MOMP), jnp.float32),
                        pltpu.VMEM((2 * n_atoms, _MOMP), jnp.float32)],
    )(idx3d, drp, cfp)

    m = mom[:, :_N_RADIAL * _MOM].reshape(n_atoms, _N_RADIAL, _MOM)
    m0 = m[:, :, 0]
    m1 = m[:, :, 1:4]
    m2 = m[:, :, 4:13].reshape(n_atoms, _N_RADIAL, 3, 3)
    m3 = m[:, :, 13:40].reshape(n_atoms, _N_RADIAL, 3, 3, 3)

    contr_1 = jnp.einsum('ari,asi->ars', m1, m1)
    contr_2 = jnp.einsum('arij,asij->ars', m2, m2)
    contr_3 = jnp.einsum('arijk,asijk->ars', m3, m3)
    contr_4 = jnp.einsum('arij,asik,atjk->arst', m2, m2, m2)
    contr_5 = jnp.einsum('ari,asj,atij->arst', m1, m1, m2)
    contr_6 = jnp.einsum('arijk,asijl,atkl->arst', m3, m3, m2)
    contr_7 = jnp.einsum('arijk,asij,atk->arst', m3, m2, m1)
    i2, j2 = _tril2(_N_RADIAL)
    i3, j3, k3 = _tril3(_N_RADIAL)
    n = _N_RADIAL
    n2 = i2.shape[0]
    gm = jnp.concatenate([
        m0,
        contr_1[:, i2, j2],
        contr_2[:, i2, j2],
        contr_3[:, i2, j2],
        contr_4[:, i3, j3, k3],
        contr_5[:, i2, j2, :].reshape(-1, n2 * n),
        contr_6[:, i2, j2, :].reshape(-1, n2 * n),
        contr_7.reshape(-1, n * n * n),
    ], axis=-1)

    feat = gm.shape[-1]
    featp = 384
    gmp = jnp.concatenate(
        [gm, jnp.zeros((n_atoms, featp - feat), dtype=jnp.float32)], axis=1)
    w0p = jnp.concatenate(
        [W0, jnp.zeros((featp - feat, W0.shape[1]), dtype=jnp.float32)], axis=0)
    w2p = jnp.concatenate(
        [W2, jnp.zeros((W2.shape[0], 127), dtype=jnp.float32)], axis=1)
    b2p = jnp.concatenate([b2, jnp.zeros((127,), dtype=jnp.float32)])

    a_blk = _A_BLK if n_atoms % _A_BLK == 0 else n_atoms
    grid_b = n_atoms // a_blk
    hid = W0.shape[1]
    mlp = functools.partial(_mlp_kernel,
                            inv0=1.0 / np.sqrt(float(feat)),
                            inv1=1.0 / np.sqrt(float(hid)))
    h = pl.pallas_call(
        mlp,
        grid=(grid_b,),
        in_specs=[
            pl.BlockSpec((a_blk, featp), lambda i: (i, 0)),
            pl.BlockSpec((featp, hid), lambda i: (0, 0)),
            pl.BlockSpec((1, hid), lambda i: (0, 0)),
            pl.BlockSpec((hid, hid), lambda i: (0, 0)),
            pl.BlockSpec((1, hid), lambda i: (0, 0)),
            pl.BlockSpec((hid, 128), lambda i: (0, 0)),
            pl.BlockSpec((1, 128), lambda i: (0, 0)),
        ],
        out_specs=pl.BlockSpec((a_blk, 128), lambda i: (i, 0)),
        out_shape=jax.ShapeDtypeStruct((n_atoms, 128), jnp.float32),
    )(gmp, w0p, b0.reshape(1, hid), W1, b1.reshape(1, hid), w2p,
      b2p.reshape(1, 128))

    out = h[:, :1]
    out = scale[Z] * out + shift[Z]
    return jnp.where((Z > 0)[:, None], out, 0.0)
